# baseline (speedup 1.0000x reference)
import functools

import jax
import jax.numpy as jnp
from jax import lax
from jax.experimental import pallas as pl
from jax.experimental.pallas import tpu as pltpu
from jax.experimental.pallas import tpu_sc as plsc

_PATCH = 8
_IDX_CHUNK = 128


@functools.lru_cache(maxsize=None)
def _make_sc_gather(num_idx: int, dim: int):
    info = plsc.get_sparse_core_info()
    nc, ns = info.num_cores, info.num_subcores
    nw = nc * ns
    rows_per_w = num_idx // nw
    chunks = rows_per_w // _IDX_CHUNK
    mesh = plsc.VectorSubcoreMesh(core_axis_name="c", subcore_axis_name="s")

    @functools.partial(
        pl.kernel,
        mesh=mesh,
        out_type=jax.ShapeDtypeStruct((num_idx, dim), jnp.float32),
        scratch_types=[
            pltpu.VMEM((chunks, _IDX_CHUNK), jnp.int32),
            pltpu.VMEM((rows_per_w, dim), jnp.float32),
            pltpu.SemaphoreType.DMA,
        ],
        compiler_params=pltpu.CompilerParams(use_tc_tiling_on_sc=False),
    )
    def gather(idx_hbm, table_hbm, out_hbm, idx_v, rows_v, sem):
        wid = lax.axis_index("s") * nc + lax.axis_index("c")
        pltpu.sync_copy(idx_hbm.at[pl.ds(wid * chunks, chunks)], idx_v)
        copies = []
        for ci in range(chunks):
            copies.append(
                pltpu.async_copy(
                    table_hbm.at[idx_v.at[ci]],
                    rows_v.at[pl.ds(ci * _IDX_CHUNK, _IDX_CHUNK)],
                    sem,
                )
            )
        for cp in copies:
            cp.wait()
        pltpu.sync_copy(rows_v, out_hbm.at[pl.ds(wid * rows_per_w, rows_per_w)])

    return gather


def _mm_body(m_ref, w0_ref, w1_ref, b_ref, o_ref):
    bm = o_ref.shape[0]
    m3 = m_ref[...].reshape(bm, 2, 128)
    a0 = m3[:, 0, :]
    a1 = m3[:, 1, :]
    o_ref[...] = (
        jnp.dot(a0, w0_ref[0], preferred_element_type=jnp.float32)
        + jnp.dot(a1, w1_ref[0], preferred_element_type=jnp.float32)
        + b_ref[...][None, :]
    )


def _tc_matmul(m2d, w2, b, bm):
    m = m2d.shape[0] // 2
    n = w2.shape[2]
    return pl.pallas_call(
        _mm_body,
        grid=(m // bm,),
        in_specs=[
            pl.BlockSpec((2 * bm, 128), lambda i: (i, 0)),
            pl.BlockSpec((1, 128, n), lambda i: (0, 0, 0)),
            pl.BlockSpec((1, 128, n), lambda i: (1, 0, 0)),
            pl.BlockSpec((n,), lambda i: (0,)),
        ],
        out_specs=pl.BlockSpec((bm, n), lambda i: (i, 0)),
        out_shape=jax.ShapeDtypeStruct((m, n), jnp.float32),
        compiler_params=pltpu.CompilerParams(
            dimension_semantics=("arbitrary",),
        ),
    )(m2d, w2, w2, b)


def kernel(bytes_flat, table, W, b):
    B, L = bytes_flat.shape
    P = _PATCH
    T = L // P
    byte_dim = table.shape[1]
    n_idx = B * T * P
    half = P * byte_dim // 2
    n_chunks = B
    ci = n_idx // n_chunks

    idx2d = bytes_flat[:, : T * P].reshape(n_idx // _IDX_CHUNK, _IDX_CHUNK)
    gather = _make_sc_gather(ci, byte_dim)
    w2 = W.reshape(2, half, -1)
    rows_per_c = idx2d.shape[0] // n_chunks
    outs = []
    for c in range(n_chunks):
        idx_c = lax.slice_in_dim(idx2d, c * rows_per_c, (c + 1) * rows_per_c)
        embs_c = gather(idx_c, table)
        m2d_c = embs_c.reshape(ci * byte_dim // half, half)
        outs.append(_tc_matmul(m2d_c, w2, b, 512))
    out = jnp.concatenate(outs, axis=0)
    return out.reshape(B, T, -1), T


# speedup vs baseline: 1.4995x; 1.4995x over previous
import functools

import jax
import jax.numpy as jnp
from jax import lax
from jax.experimental import pallas as pl
from jax.experimental.pallas import tpu as pltpu
from jax.experimental.pallas import tpu_sc as plsc

_PATCH = 8
_IDX_CHUNK = 128


@functools.lru_cache(maxsize=None)
def _make_sc_gather(num_idx: int, dim: int):
    info = plsc.get_sparse_core_info()
    nc, ns = info.num_cores, info.num_subcores
    nw = nc * ns
    rows_per_w = num_idx // nw
    chunks = rows_per_w // _IDX_CHUNK
    mesh = plsc.VectorSubcoreMesh(core_axis_name="c", subcore_axis_name="s")

    @functools.partial(
        pl.kernel,
        mesh=mesh,
        out_type=jax.ShapeDtypeStruct((num_idx, dim), jnp.int32),
        scratch_types=[
            pltpu.VMEM((chunks, _IDX_CHUNK), jnp.int32),
            pltpu.VMEM((rows_per_w, dim), jnp.int32),
            pltpu.SemaphoreType.DMA,
        ],
        compiler_params=pltpu.CompilerParams(use_tc_tiling_on_sc=False),
    )
    def gather(idx_hbm, table_hbm, out_hbm, idx_v, rows_v, sem):
        wid = lax.axis_index("s") * nc + lax.axis_index("c")
        pltpu.sync_copy(idx_hbm.at[pl.ds(wid * chunks, chunks)], idx_v)
        copies = []
        for ci in range(chunks):
            copies.append(
                pltpu.async_copy(
                    table_hbm.at[idx_v.at[ci]],
                    rows_v.at[pl.ds(ci * _IDX_CHUNK, _IDX_CHUNK)],
                    sem,
                )
            )
        for cp in copies:
            cp.wait()
        pltpu.sync_copy(rows_v, out_hbm.at[pl.ds(wid * rows_per_w, rows_per_w)])

    return gather


def _mm_body(m_ref, w0_ref, w1_ref, b_ref, o_ref):
    bm, kw = m_ref.shape
    xb = pltpu.bitcast(m_ref[...], jnp.bfloat16)
    x3 = xb.reshape(bm, 2, kw)
    a0 = x3[:, 0, :]
    a1 = x3[:, 1, :]
    o_ref[...] = (
        jnp.dot(a0, w0_ref[0], preferred_element_type=jnp.float32)
        + jnp.dot(a1, w1_ref[0], preferred_element_type=jnp.float32)
        + b_ref[...][None, :]
    )


def _tc_matmul(m2d, w2, b, bm):
    m, kw = m2d.shape
    n = w2.shape[2]
    return pl.pallas_call(
        _mm_body,
        grid=(m // bm,),
        in_specs=[
            pl.BlockSpec((bm, kw), lambda i: (i, 0)),
            pl.BlockSpec((1, kw, n), lambda i: (0, 0, 0)),
            pl.BlockSpec((1, kw, n), lambda i: (1, 0, 0)),
            pl.BlockSpec((n,), lambda i: (0,)),
        ],
        out_specs=pl.BlockSpec((bm, n), lambda i: (i, 0)),
        out_shape=jax.ShapeDtypeStruct((m, n), jnp.float32),
        compiler_params=pltpu.CompilerParams(
            dimension_semantics=("arbitrary",),
        ),
    )(m2d, w2, w2, b)


def kernel(bytes_flat, table, W, b):
    B, L = bytes_flat.shape
    P = _PATCH
    T = L // P
    byte_dim = table.shape[1]
    n_idx = B * T * P
    dim_w = byte_dim // 2

    idx2d = bytes_flat[:, : T * P].reshape(n_idx // _IDX_CHUNK, _IDX_CHUNK)
    table_pk = lax.bitcast_convert_type(
        table.astype(jnp.bfloat16).reshape(table.shape[0], dim_w, 2),
        jnp.int32,
    )
    gather = _make_sc_gather(n_idx, dim_w)
    embs = gather(idx2d, table_pk)

    m2d = embs.reshape(B * T, P * dim_w)
    w_bf = W.astype(jnp.bfloat16)
    w2 = jnp.stack([w_bf[0::2], w_bf[1::2]])
    out = _tc_matmul(m2d, w2, b, 512)
    return out.reshape(B, T, -1), T


# speedup vs baseline: 1.5897x; 1.0602x over previous
import functools

import jax
import jax.numpy as jnp
from jax import lax
from jax.experimental import pallas as pl
from jax.experimental.pallas import tpu as pltpu
from jax.experimental.pallas import tpu_sc as plsc

_PATCH = 8
_IDX_CHUNK = 128


@functools.lru_cache(maxsize=None)
def _make_sc_gather(num_idx: int, dim: int):
    info = plsc.get_sparse_core_info()
    nc, ns = info.num_cores, info.num_subcores
    nw = nc * ns
    rows_per_w = num_idx // nw
    chunks = rows_per_w // _IDX_CHUNK
    mesh = plsc.VectorSubcoreMesh(core_axis_name="c", subcore_axis_name="s")

    @functools.partial(
        pl.kernel,
        mesh=mesh,
        out_type=jax.ShapeDtypeStruct((num_idx, dim), jnp.int32),
        scratch_types=[
            pltpu.VMEM((chunks, _IDX_CHUNK), jnp.int32),
            pltpu.VMEM((rows_per_w, dim), jnp.int32),
            pltpu.SemaphoreType.DMA,
            pltpu.SemaphoreType.DMA,
        ],
        compiler_params=pltpu.CompilerParams(use_tc_tiling_on_sc=False),
    )
    def gather(idx_hbm, table_hbm, out_hbm, idx_v, rows_v, sem_g, sem_w):
        wid = lax.axis_index("s") * nc + lax.axis_index("c")
        base = wid * rows_per_w
        pltpu.sync_copy(idx_hbm.at[pl.ds(wid * chunks, chunks)], idx_v)

        def _gather_chunk(ci):
            return pltpu.make_async_copy(
                table_hbm.at[idx_v.at[ci]],
                rows_v.at[pl.ds(ci * _IDX_CHUNK, _IDX_CHUNK)],
                sem_g,
            )

        def _write_chunk(ci):
            return pltpu.make_async_copy(
                rows_v.at[pl.ds(ci * _IDX_CHUNK, _IDX_CHUNK)],
                out_hbm.at[pl.ds(base + ci * _IDX_CHUNK, _IDX_CHUNK)],
                sem_w,
            )

        lead = 4
        for ci in range(lead):
            _gather_chunk(ci).start()

        @pl.loop(0, chunks)
        def _pipeline(ci):
            @pl.when(ci < chunks - lead)
            def _():
                _gather_chunk(ci + lead).start()

            _gather_chunk(ci).wait()
            _write_chunk(ci).start()

        @pl.loop(0, chunks)
        def _drain(ci):
            _write_chunk(ci).wait()

    return gather


def _mm_body(m_ref, w0_ref, w1_ref, b_ref, o_ref):
    bm, kw = m_ref.shape
    xb = pltpu.bitcast(m_ref[...], jnp.bfloat16)
    x3 = xb.reshape(bm, 2, kw)
    a0 = x3[:, 0, :]
    a1 = x3[:, 1, :]
    o_ref[...] = (
        jnp.dot(a0, w0_ref[0], preferred_element_type=jnp.float32)
        + jnp.dot(a1, w1_ref[0], preferred_element_type=jnp.float32)
        + b_ref[...][None, :]
    )


def _tc_matmul(m2d, w2, b, bm):
    m, kw = m2d.shape
    n = w2.shape[2]
    return pl.pallas_call(
        _mm_body,
        grid=(m // bm,),
        in_specs=[
            pl.BlockSpec((bm, kw), lambda i: (i, 0)),
            pl.BlockSpec((1, kw, n), lambda i: (0, 0, 0)),
            pl.BlockSpec((1, kw, n), lambda i: (1, 0, 0)),
            pl.BlockSpec((n,), lambda i: (0,)),
        ],
        out_specs=pl.BlockSpec((bm, n), lambda i: (i, 0)),
        out_shape=jax.ShapeDtypeStruct((m, n), jnp.float32),
        compiler_params=pltpu.CompilerParams(
            dimension_semantics=("arbitrary",),
        ),
    )(m2d, w2, w2, b)


def kernel(bytes_flat, table, W, b):
    B, L = bytes_flat.shape
    P = _PATCH
    T = L // P
    byte_dim = table.shape[1]
    n_idx = B * T * P
    dim_w = byte_dim // 2

    idx2d = bytes_flat[:, : T * P].reshape(n_idx // _IDX_CHUNK, _IDX_CHUNK)
    table_pk = lax.bitcast_convert_type(
        table.astype(jnp.bfloat16).reshape(table.shape[0], dim_w, 2),
        jnp.int32,
    )
    gather = _make_sc_gather(n_idx, dim_w)
    embs = gather(idx2d, table_pk)

    m2d = embs.reshape(B * T, P * dim_w)
    w_bf = W.astype(jnp.bfloat16)
    w2 = jnp.stack([w_bf[0::2], w_bf[1::2]])
    out = _tc_matmul(m2d, w2, b, 1024)
    return out.reshape(B, T, -1), T
